# baseline (speedup 1.0000x reference)
import functools

import jax
import jax.numpy as jnp
from jax import lax
from jax.experimental import pallas as pl
from jax.experimental.pallas import tpu as pltpu
from jax.experimental.pallas import tpu_sc as plsc

N_HEADS = 16
MAX_DIST = 128
NREL = 2 * MAX_DIST + 1
SEQ = 2048
NSHIFT = 128
WW = SEQ + (NSHIFT // 8 - 1) * NSHIFT
E8W = WW + NSHIFT * 2
EBASE = E8W + 7
FIRE = 8


def _build_e8(relative_bias):
    t = relative_bias
    left = jnp.broadcast_to(t[:, :1], (N_HEADS, SEQ - 1 - MAX_DIST))
    right = jnp.broadcast_to(t[:, -1:], (N_HEADS, EBASE - (SEQ - 1 - MAX_DIST) - NREL))
    e = jnp.concatenate([left, t, right], axis=1)
    return jnp.stack([e[:, 7 - s : 7 - s + E8W] for s in range(8)], axis=1)


def _tc_build_w(e8_ref, w_ref):
    a = pl.program_id(1)
    for av in range(NSHIFT // 8):
        off = (NSHIFT - 8) - 8 * av

        @pl.when(a == av)
        def _():
            w_ref[0, :, :] = e8_ref[0, :, off : off + WW]


def _sc_emit(w_hbm, out_hbm, sem):
    c = lax.axis_index("c")
    s = lax.axis_index("s")
    wid = s * 2 + c
    h = wid // 2
    b_base = (wid % 2) * 8

    def body(bb, carry):
        b = b_base + bb
        copies = []
        for p in range(FIRE):
            cp = pltpu.make_async_copy(
                w_hbm.at[h, pl.ds(16 * p, 16), pl.ds(NSHIFT * (15 - b), SEQ)],
                out_hbm.at[h, pl.ds(NSHIFT * b + 16 * p, 16)],
                sem,
            )
            cp.start()
            copies.append(cp)
        for cp in copies:
            cp.wait()
        return carry

    lax.fori_loop(0, 8, body, 0)


def kernel(seq_len, relative_bias):
    del seq_len
    e8 = _build_e8(relative_bias)

    w = pl.pallas_call(
        _tc_build_w,
        out_shape=jax.ShapeDtypeStruct((N_HEADS, NSHIFT, WW), jnp.float32),
        grid=(N_HEADS, NSHIFT // 8),
        in_specs=[pl.BlockSpec((1, 8, E8W), lambda h, a: (h, 0, 0))],
        out_specs=pl.BlockSpec((1, 8, WW), lambda h, a: (h, a, 0)),
        compiler_params=pltpu.CompilerParams(
            dimension_semantics=("parallel", "parallel"),
        ),
    )(e8)

    mesh = plsc.VectorSubcoreMesh(core_axis_name="c", subcore_axis_name="s")
    run = functools.partial(
        pl.kernel,
        mesh=mesh,
        out_type=jax.ShapeDtypeStruct((N_HEADS, SEQ, SEQ), jnp.float32),
        scratch_types=[pltpu.SemaphoreType.DMA],
    )(_sc_emit)
    return run(w)


# speedup vs baseline: 35.5112x; 35.5112x over previous
import functools

import jax
import jax.numpy as jnp
from jax import lax
from jax.experimental import pallas as pl
from jax.experimental.pallas import tpu as pltpu
from jax.experimental.pallas import tpu_sc as plsc

N_HEADS = 16
MAX_DIST = 128
NREL = 2 * MAX_DIST + 1
SEQ = 2048
NSHIFT = 128
WW = SEQ + (NSHIFT // 8 - 1) * NSHIFT
E8W = WW + NSHIFT * 2
EBASE = E8W + 7
FIRE = 8


def _build_e8(relative_bias):
    t = relative_bias
    left = jnp.broadcast_to(t[:, :1], (N_HEADS, SEQ - 1 - MAX_DIST))
    right = jnp.broadcast_to(t[:, -1:], (N_HEADS, EBASE - (SEQ - 1 - MAX_DIST) - NREL))
    e = jnp.concatenate([left, t, right], axis=1)
    return jnp.stack([e[:, 7 - s : 7 - s + E8W] for s in range(8)], axis=1)


def _tc_build_w(e8_ref, w_ref):
    a = pl.program_id(1)
    for av in range(NSHIFT // 8):
        off = (NSHIFT - 8) - 8 * av

        @pl.when(a == av)
        def _():
            w_ref[0, :, :] = e8_ref[0, :, off : off + WW]


def _sc_emit(w_hbm, out_hbm, buf0, buf1, rsem, wsem):
    c = lax.axis_index("c")
    s = lax.axis_index("s")
    wid = s * 2 + c
    h = wid // 2
    p0 = 4 * (wid % 2)

    def read_cp(u, buf):
        return pltpu.make_async_copy(
            w_hbm.at[h, pl.ds(16 * (p0 + u), 16), :], buf, rsem
        )

    read_cp(0, buf0).start()
    for u in range(4):
        buf, nbuf = (buf0, buf1) if u % 2 == 0 else (buf1, buf0)
        read_cp(u, buf).wait()
        cps = []
        for b in range(16):
            cp = pltpu.make_async_copy(
                buf.at[:, pl.ds(NSHIFT * (15 - b), SEQ)],
                out_hbm.at[h, pl.ds(NSHIFT * b + 16 * (p0 + u), 16)],
                wsem,
            )
            cp.start()
            cps.append(cp)
        if u < 3:
            read_cp(u + 1, nbuf).start()
        for cp in cps:
            cp.wait()


def kernel(seq_len, relative_bias):
    del seq_len
    e8 = _build_e8(relative_bias)

    w = pl.pallas_call(
        _tc_build_w,
        out_shape=jax.ShapeDtypeStruct((N_HEADS, NSHIFT, WW), jnp.float32),
        grid=(N_HEADS, NSHIFT // 8),
        in_specs=[pl.BlockSpec((1, 8, E8W), lambda h, a: (h, 0, 0))],
        out_specs=pl.BlockSpec((1, 8, WW), lambda h, a: (h, a, 0)),
        compiler_params=pltpu.CompilerParams(
            dimension_semantics=("parallel", "parallel"),
        ),
    )(e8)

    mesh = plsc.VectorSubcoreMesh(core_axis_name="c", subcore_axis_name="s")
    run = functools.partial(
        pl.kernel,
        mesh=mesh,
        out_type=jax.ShapeDtypeStruct((N_HEADS, SEQ, SEQ), jnp.float32),
        scratch_types=[
            pltpu.VMEM((16, WW), jnp.float32),
            pltpu.VMEM((16, WW), jnp.float32),
            pltpu.SemaphoreType.DMA,
            pltpu.SemaphoreType.DMA,
        ],
    )(_sc_emit)
    return run(w)


# speedup vs baseline: 57.5841x; 1.6216x over previous
import functools

import jax
import jax.numpy as jnp
from jax import lax
from jax.experimental import pallas as pl
from jax.experimental.pallas import tpu as pltpu
from jax.experimental.pallas import tpu_sc as plsc

N_HEADS = 16
MAX_DIST = 128
NREL = 2 * MAX_DIST + 1
SEQ = 2048
NSHIFT = 128
WW = SEQ + (NSHIFT // 8 - 1) * NSHIFT
E8W = WW + NSHIFT * 2
EBASE = E8W + 7
FIRE = 8


def _build_e8(relative_bias):
    t = relative_bias
    left = jnp.broadcast_to(t[:, :1], (N_HEADS, SEQ - 1 - MAX_DIST))
    right = jnp.broadcast_to(t[:, -1:], (N_HEADS, EBASE - (SEQ - 1 - MAX_DIST) - NREL))
    e = jnp.concatenate([left, t, right], axis=1)
    return jnp.stack([e[:, 7 - s : 7 - s + E8W] for s in range(8)], axis=1)


def _tc_build_w(e8_ref, w_ref):
    for a in range(NSHIFT // 8):
        off = (NSHIFT - 8) - 8 * a
        w_ref[0, 8 * a : 8 * a + 8, :] = e8_ref[0, :, off : off + WW]


def _sc_emit(w_hbm, out_hbm, buf0, buf1, rsem, wsem):
    c = lax.axis_index("c")
    s = lax.axis_index("s")
    wid = s * 2 + c
    h = wid // 2
    p0 = 4 * (wid % 2)

    def read_cp(u, buf):
        return pltpu.make_async_copy(
            w_hbm.at[h, pl.ds(16 * (p0 + u), 16), :], buf, rsem
        )

    read_cp(0, buf0).start()
    for u in range(4):
        buf, nbuf = (buf0, buf1) if u % 2 == 0 else (buf1, buf0)
        read_cp(u, buf).wait()
        cps = []
        for b in range(16):
            cp = pltpu.make_async_copy(
                buf.at[:, pl.ds(NSHIFT * (15 - b), SEQ)],
                out_hbm.at[h, pl.ds(NSHIFT * b + 16 * (p0 + u), 16)],
                wsem,
            )
            cp.start()
            cps.append(cp)
        if u < 3:
            read_cp(u + 1, nbuf).start()
        for cp in cps:
            cp.wait()


def kernel(seq_len, relative_bias):
    del seq_len
    e8 = _build_e8(relative_bias)

    w = pl.pallas_call(
        _tc_build_w,
        out_shape=jax.ShapeDtypeStruct((N_HEADS, NSHIFT, WW), jnp.float32),
        grid=(N_HEADS,),
        in_specs=[pl.BlockSpec((1, 8, E8W), lambda h: (h, 0, 0))],
        out_specs=pl.BlockSpec((1, NSHIFT, WW), lambda h: (h, 0, 0)),
        compiler_params=pltpu.CompilerParams(
            dimension_semantics=("parallel",),
        ),
    )(e8)

    mesh = plsc.VectorSubcoreMesh(core_axis_name="c", subcore_axis_name="s")
    run = functools.partial(
        pl.kernel,
        mesh=mesh,
        out_type=jax.ShapeDtypeStruct((N_HEADS, SEQ, SEQ), jnp.float32),
        scratch_types=[
            pltpu.VMEM((16, WW), jnp.float32),
            pltpu.VMEM((16, WW), jnp.float32),
            pltpu.SemaphoreType.DMA,
            pltpu.SemaphoreType.DMA,
        ],
    )(_sc_emit)
    return run(w)


# speedup vs baseline: 61.6484x; 1.0706x over previous
import functools

import jax
import jax.numpy as jnp
from jax import lax
from jax.experimental import pallas as pl
from jax.experimental.pallas import tpu as pltpu
from jax.experimental.pallas import tpu_sc as plsc

N_HEADS = 16
MAX_DIST = 128
NREL = 2 * MAX_DIST + 1
SEQ = 2048
NSHIFT = 128
WW = SEQ + (NSHIFT // 8 - 1) * NSHIFT
EPAD = 4352


def _build_e(relative_bias):
    t = relative_bias
    left = jnp.broadcast_to(t[:, :1], (N_HEADS, SEQ - 1 - MAX_DIST))
    right = jnp.broadcast_to(t[:, -1:], (N_HEADS, EPAD - (SEQ - 1 - MAX_DIST) - NREL))
    return jnp.concatenate([left, t, right], axis=1)[:, None, :]


def _tc_build_w(e_ref, w_ref):
    row = e_ref[0, 0, :]
    bc = jnp.broadcast_to(row[None, :], (8, EPAD))
    for a in range(NSHIFT // 8):
        w_ref[0, 8 * a : 8 * a + 8, :] = pltpu.roll(
            bc, (EPAD - 127) + 8 * a, 1, stride=1, stride_axis=0
        )[:, :WW]


def _sc_emit(w_hbm, out_hbm, buf0, buf1, rsem, wsem0, wsem1):
    c = lax.axis_index("c")
    s = lax.axis_index("s")
    wid = s * 2 + c
    h = wid // 2
    p0 = 4 * (wid % 2)

    def read_cp(u, buf):
        return pltpu.make_async_copy(
            w_hbm.at[h, pl.ds(16 * (p0 + u), 16), :], buf, rsem
        )

    def write_cps(u, buf, wsem):
        return [
            pltpu.make_async_copy(
                buf.at[:, pl.ds(NSHIFT * (15 - b), SEQ)],
                out_hbm.at[h, pl.ds(NSHIFT * b + 16 * (p0 + u), 16)],
                wsem,
            )
            for b in range(16)
        ]

    read_cp(0, buf0).start()
    for u in range(4):
        buf, nbuf = (buf0, buf1) if u % 2 == 0 else (buf1, buf0)
        wsem = wsem0 if u % 2 == 0 else wsem1
        read_cp(u, buf).wait()
        for cp in write_cps(u, buf, wsem):
            cp.start()
        if u >= 1:
            for cp in write_cps(u - 1, nbuf, wsem0 if u % 2 == 1 else wsem1):
                cp.wait()
        if u < 3:
            read_cp(u + 1, nbuf).start()
    for cp in write_cps(3, buf1, wsem1):
        cp.wait()


def kernel(seq_len, relative_bias):
    del seq_len
    e = _build_e(relative_bias)

    w = pl.pallas_call(
        _tc_build_w,
        out_shape=jax.ShapeDtypeStruct((N_HEADS, NSHIFT, WW), jnp.float32),
        grid=(N_HEADS,),
        in_specs=[pl.BlockSpec((1, 1, EPAD), lambda h: (h, 0, 0))],
        out_specs=pl.BlockSpec((1, NSHIFT, WW), lambda h: (h, 0, 0)),
        compiler_params=pltpu.CompilerParams(
            dimension_semantics=("parallel",),
        ),
    )(e)

    mesh = plsc.VectorSubcoreMesh(core_axis_name="c", subcore_axis_name="s")
    run = functools.partial(
        pl.kernel,
        mesh=mesh,
        out_type=jax.ShapeDtypeStruct((N_HEADS, SEQ, SEQ), jnp.float32),
        scratch_types=[
            pltpu.VMEM((16, WW), jnp.float32),
            pltpu.VMEM((16, WW), jnp.float32),
            pltpu.SemaphoreType.DMA,
            pltpu.SemaphoreType.DMA,
            pltpu.SemaphoreType.DMA,
        ],
    )(_sc_emit)
    return run(w)
